# baseline (speedup 1.0000x reference)
import functools

import jax
import jax.numpy as jnp
from jax import lax
from jax.experimental import pallas as pl
from jax.experimental.pallas import tpu as pltpu
from jax.experimental.pallas import tpu_sc as plsc

E = 8
H = 1024
FF = 512
T = 2048
BLK = 128
N_ITEMS = 2 * T
P_MAX = N_ITEMS + E * BLK
G_PAD = P_MAX // BLK
G_TOT = G_PAD + T // BLK
P_TOT = P_MAX + T
NEG = -1e30

NC = 2
NS = 16
NW = NC * NS
IPW = N_ITEMS // NW


def _sigmoid(x):
    return 1.0 / (1.0 + jnp.exp(-x))


def _mesh():
    return plsc.VectorSubcoreMesh(core_axis_name="c", subcore_axis_name="s")


def _wid():
    return lax.axis_index("s") * NC + lax.axis_index("c")


def _shf_init(shf, zero):
    shf[pl.ds(0, 16)] = zero
    shf[pl.ds(32, 16)] = zero


def _shift_up(v, shf, k):
    shf[pl.ds(16, 16)] = v
    return shf[pl.ds(16 - k, 16)]


def _shift_down(v, shf, k):
    shf[pl.ds(16, 16)] = v
    return shf[pl.ds(16 + k, 16)]


def _scan16(v, shf):
    for k in (1, 2, 4, 8):
        v = v + _shift_up(v, shf, k)
    return v


def _propagate0(t, shf):
    for k in (1, 2, 4, 8):
        t = t + _shift_up(t, shf, k)
    return t


def _splat_last(v, shf, lane):
    r = lax.rev(v, (0,))
    return _propagate0(jnp.where(lane == 0, r, jnp.zeros((16,), r.dtype)), shf)


def _splat_at(v, shf, lane, e):
    t = _shift_down(v, shf, e) if e else v
    return _propagate0(jnp.where(lane == 0, t, jnp.zeros((16,), t.dtype)), shf)


def _router_body(x_ref, gwp_ref, i_ref, f_ref):
    lt = lax.dot_general(gwp_ref[...], x_ref[...], (((1,), (1,)), ((), ())),
                         preferred_element_type=jnp.float32)
    row = lax.broadcasted_iota(jnp.int32, lt.shape, 0)
    lm = jnp.where(row < E, lt, NEG)
    m0 = jnp.max(lm, axis=0, keepdims=True)
    a0 = jnp.min(jnp.where(lm == m0, row, 999), axis=0, keepdims=True)
    lm2 = jnp.where(row == a0, NEG, lm)
    m1 = jnp.max(lm2, axis=0, keepdims=True)
    a1 = jnp.min(jnp.where(lm2 == m1, row, 999), axis=0, keepdims=True)
    w0 = _sigmoid(m0 - m1)
    w1 = _sigmoid(m1 - m0)
    sg = _sigmoid(lt[E:E + 1, :])
    r8 = lax.broadcasted_iota(jnp.int32, (8, a0.shape[1]), 0)
    i_ref[...] = jnp.where(r8 == 0, a0, jnp.where(r8 == 1, a1, 0))
    f_ref[...] = jnp.where(r8 == 0, w0,
                           jnp.where(r8 == 1, w1,
                                     jnp.where(r8 == 2, sg, 0.0)))


def _router(x, gwp):
    return pl.pallas_call(
        _router_body,
        grid=(T // BLK,),
        in_specs=[
            pl.BlockSpec((BLK, H), lambda g: (g, 0)),
            pl.BlockSpec((128, H), lambda g: (0, 0)),
        ],
        out_specs=[
            pl.BlockSpec((8, BLK), lambda g: (0, g)),
            pl.BlockSpec((8, BLK), lambda g: (0, g)),
        ],
        out_shape=[
            jax.ShapeDtypeStruct((8, T), jnp.int32),
            jax.ShapeDtypeStruct((8, T), jnp.float32),
        ],
    )(x, gwp)


def _bin_a_kernel(ids_hbm, lane_hbm, rank_hbm, cnt_hbm,
                  eids_v, rank_v, cnt_v, shf_v, lane_v):
    wid = _wid()
    flatbase = wid * IPW
    pltpu.sync_copy(lane_hbm, lane_v)
    lane = lane_v[pl.ds(0, 16)]
    zero = jnp.zeros((16,), jnp.int32)
    _shf_init(shf_v, zero)
    pltpu.sync_copy(ids_hbm.at[flatbase // T, pl.ds(flatbase % T, IPW)], eids_v)
    nv = IPW // 16
    ranks = [zero for _ in range(nv)]
    counts = zero
    for e in range(E):
        run = zero
        for v in range(nv):
            ev = eids_v[pl.ds(v * 16, 16)]
            m = ev == e
            c = _scan16(jnp.where(m, 1, 0), shf_v)
            ranks[v] = jnp.where(m, run + c - 1, ranks[v])
            run = run + _splat_last(c, shf_v, lane)
        counts = jnp.where(lane == e, run, counts)
    for v in range(nv):
        rank_v[pl.ds(v * 16, 16)] = ranks[v]
    cnt_v[pl.ds(0, 16)] = counts
    pltpu.sync_copy(rank_v, rank_hbm.at[pl.ds(flatbase, IPW)])
    pltpu.sync_copy(cnt_v, cnt_hbm.at[pl.ds(wid * 16, 16)])


def _bin_a(ids, lane16):
    f = functools.partial(
        pl.kernel,
        mesh=_mesh(),
        out_type=[
            jax.ShapeDtypeStruct((N_ITEMS,), jnp.int32),
            jax.ShapeDtypeStruct((NW * 16,), jnp.int32),
        ],
        scratch_types=[
            pltpu.VMEM((IPW,), jnp.int32),
            pltpu.VMEM((IPW,), jnp.int32),
            pltpu.VMEM((16,), jnp.int32),
            pltpu.VMEM((48,), jnp.int32),
            pltpu.VMEM((16,), jnp.int32),
        ],
    )(_bin_a_kernel)
    return f(ids, lane16)


def _bin_b_kernel(ids_hbm, lane_hbm, rank_hbm, cnt_hbm, pos_hbm, bexp_hbm,
                  eids_v, rank_v, pos_v, allc_v, shf_v, lane_v, pfx_v, bexp_v):
    wid = _wid()
    flatbase = wid * IPW
    pltpu.sync_copy(lane_hbm, lane_v)
    lane = lane_v[pl.ds(0, 16)]
    zero = jnp.zeros((16,), jnp.int32)
    _shf_init(shf_v, zero)
    pltpu.sync_copy(ids_hbm.at[flatbase // T, pl.ds(flatbase % T, IPW)], eids_v)
    pltpu.sync_copy(rank_hbm.at[pl.ds(flatbase, IPW)], rank_v)
    pltpu.sync_copy(cnt_hbm, allc_v)
    pfx_v[pl.ds(0, 16)] = zero
    total = zero
    for w in range(NW):
        rowv = allc_v[pl.ds(w * 16, 16)]
        total = total + rowv

        @pl.when(w < wid)
        def _acc(rowv=rowv):
            pfx_v[pl.ds(0, 16)] = pfx_v[pl.ds(0, 16)] + rowv

    padded = jnp.bitwise_and(total + (BLK - 1), -BLK)
    off = _scan16(padded, shf_v) - padded
    base = off + pfx_v[pl.ds(0, 16)]
    base_splats = [_splat_at(base, shf_v, lane, e) for e in range(E)]
    nv = IPW // 16
    for v in range(nv):
        ev = eids_v[pl.ds(v * 16, 16)]
        b = zero
        for e in range(E):
            b = jnp.where(ev == e, base_splats[e], b)
        pos_v[pl.ds(v * 16, 16)] = b + rank_v[pl.ds(v * 16, 16)]
    pltpu.sync_copy(pos_v, pos_hbm.at[pl.ds(flatbase, IPW)])

    @pl.when(wid == 0)
    def _sched():
        off_splats = [_splat_at(off, shf_v, lane, e) for e in range(E)]
        for cq in range(4):
            bidx = lane + cq * 16
            bstart = bidx * BLK
            acc = jnp.full((16,), -1, jnp.int32)
            for e in range(E):
                acc = acc + jnp.where(bstart >= off_splats[e], 1, 0)
            acc = jnp.where(bidx >= G_PAD, E, acc)
            bexp_v[pl.ds(cq * 16, 16)] = acc
        pltpu.sync_copy(bexp_v, bexp_hbm)


def _bin_b(ids, lane16, rank, cnt):
    f = functools.partial(
        pl.kernel,
        mesh=_mesh(),
        out_type=[
            jax.ShapeDtypeStruct((N_ITEMS,), jnp.int32),
            jax.ShapeDtypeStruct((64,), jnp.int32),
        ],
        scratch_types=[
            pltpu.VMEM((IPW,), jnp.int32),
            pltpu.VMEM((IPW,), jnp.int32),
            pltpu.VMEM((IPW,), jnp.int32),
            pltpu.VMEM((NW * 16,), jnp.int32),
            pltpu.VMEM((48,), jnp.int32),
            pltpu.VMEM((16,), jnp.int32),
            pltpu.VMEM((16,), jnp.int32),
            pltpu.VMEM((64,), jnp.int32),
        ],
    )(_bin_b_kernel)
    return f(ids, lane16, rank, cnt)


DCHUNK = 32


def _dispatch_kernel(x_hbm, pos_hbm, xs_hbm, idx_v, rows_v, sem):
    wid = _wid()
    flatbase = wid * IPW
    t0 = flatbase % T
    for cc in range(IPW // DCHUNK):
        pltpu.sync_copy(pos_hbm.at[pl.ds(flatbase + cc * DCHUNK, DCHUNK)], idx_v)
        pltpu.sync_copy(x_hbm.at[pl.ds(t0 + cc * DCHUNK, DCHUNK)], rows_v)
        pltpu.async_copy(rows_v, xs_hbm.at[idx_v], sem).wait()


def _dispatch(x, pos):
    f = functools.partial(
        pl.kernel,
        mesh=_mesh(),
        out_type=jax.ShapeDtypeStruct((P_MAX, H // 2), jnp.int32),
        scratch_types=[
            pltpu.VMEM((DCHUNK,), jnp.int32),
            pltpu.VMEM((DCHUNK, H // 2), jnp.int32),
            pltpu.SemaphoreType.DMA,
        ],
    )(_dispatch_kernel)
    return f(x, pos)


def _grouped_body(s_ref, xs_ref, x_ref, wg_ref, wu_ref, wd_ref, y_ref):
    g = pl.program_id(0)
    xb = jnp.where(g < G_PAD, xs_ref[...], x_ref[...])
    hg = lax.dot_general(xb, wg_ref[0], (((1,), (1,)), ((), ())),
                         preferred_element_type=jnp.float32)
    hu = lax.dot_general(xb, wu_ref[0], (((1,), (1,)), ((), ())),
                         preferred_element_type=jnp.float32)
    ha = (hg * _sigmoid(hg) * hu).astype(jnp.bfloat16)
    y_ref[...] = lax.dot_general(ha, wd_ref[0], (((1,), (1,)), ((), ())),
                                 preferred_element_type=jnp.float32)


def _grouped_mlp(x_sorted, x, wg_ext, wu_ext, wd_ext, bexp):
    grid_spec = pltpu.PrefetchScalarGridSpec(
        num_scalar_prefetch=1,
        grid=(G_TOT,),
        in_specs=[
            pl.BlockSpec((BLK, H), lambda g, s: (jnp.minimum(g, G_PAD - 1), 0)),
            pl.BlockSpec((BLK, H), lambda g, s: (jnp.maximum(g - G_PAD, 0), 0)),
            pl.BlockSpec((1, FF, H), lambda g, s: (s[g], 0, 0)),
            pl.BlockSpec((1, FF, H), lambda g, s: (s[g], 0, 0)),
            pl.BlockSpec((1, H, FF), lambda g, s: (s[g], 0, 0)),
        ],
        out_specs=pl.BlockSpec((BLK, H), lambda g, s: (g, 0)),
    )
    return pl.pallas_call(
        _grouped_body,
        grid_spec=grid_spec,
        out_shape=jax.ShapeDtypeStruct((P_TOT, H), jnp.float32),
    )(bexp, x_sorted, x, wg_ext, wu_ext, wd_ext)


TOK_PER_CTILE = T // NW
CCHUNK = 8


def _combine_kernel(pos_hbm, f_hbm, lane_hbm, y_hbm, out_hbm,
                    p0_v, p1_v, w0_v, w1_v, sg_v, y0_v, y1_v, ys_v, o_v,
                    shf_v, lane_v, sem):
    wid = _wid()
    pltpu.sync_copy(lane_hbm, lane_v)
    lane = lane_v[pl.ds(0, 16)]
    _shf_init(shf_v, jnp.zeros((16,), jnp.float32))

    def body(cc, carry):
        t0 = wid * TOK_PER_CTILE + cc * CCHUNK
        pltpu.sync_copy(pos_hbm.at[pl.ds(t0, CCHUNK)], p0_v)
        pltpu.sync_copy(pos_hbm.at[pl.ds(T + t0, CCHUNK)], p1_v)
        pltpu.sync_copy(f_hbm.at[0, pl.ds(t0, CCHUNK)], w0_v.at[pl.ds(0, CCHUNK)])
        pltpu.sync_copy(f_hbm.at[1, pl.ds(t0, CCHUNK)], w1_v.at[pl.ds(0, CCHUNK)])
        pltpu.sync_copy(f_hbm.at[2, pl.ds(t0, CCHUNK)], sg_v.at[pl.ds(0, CCHUNK)])
        pltpu.async_copy(y_hbm.at[p0_v], y0_v, sem).wait()
        pltpu.async_copy(y_hbm.at[p1_v], y1_v, sem).wait()
        pltpu.sync_copy(y_hbm.at[pl.ds(P_MAX + t0, CCHUNK)], ys_v)
        w0a = w0_v[pl.ds(0, 16)]
        w1a = w1_v[pl.ds(0, 16)]
        sga = sg_v[pl.ds(0, 16)]
        for tt in range(CCHUNK):
            w0s = _splat_at(w0a, shf_v, lane, tt)
            w1s = _splat_at(w1a, shf_v, lane, tt)
            sgs = _splat_at(sga, shf_v, lane, tt)
            for j in range(H // 16):
                sl = pl.ds(j * 16, 16)
                o_v[tt, sl] = (sgs * ys_v[tt, sl] + w0s * y0_v[tt, sl]
                               + w1s * y1_v[tt, sl])
        pltpu.sync_copy(o_v, out_hbm.at[pl.ds(t0, CCHUNK)])
        return carry

    lax.fori_loop(0, TOK_PER_CTILE // CCHUNK, body, 0)


def _combine(pos, fvals, lane16, y):
    f = functools.partial(
        pl.kernel,
        mesh=_mesh(),
        out_type=jax.ShapeDtypeStruct((T, H), jnp.float32),
        scratch_types=[
            pltpu.VMEM((CCHUNK,), jnp.int32),
            pltpu.VMEM((CCHUNK,), jnp.int32),
            pltpu.VMEM((16,), jnp.float32),
            pltpu.VMEM((16,), jnp.float32),
            pltpu.VMEM((16,), jnp.float32),
            pltpu.VMEM((CCHUNK, H), jnp.float32),
            pltpu.VMEM((CCHUNK, H), jnp.float32),
            pltpu.VMEM((CCHUNK, H), jnp.float32),
            pltpu.VMEM((CCHUNK, H), jnp.float32),
            pltpu.VMEM((48,), jnp.float32),
            pltpu.VMEM((16,), jnp.int32),
            pltpu.SemaphoreType.DMA,
        ],
    )(_combine_kernel)
    return f(pos, fvals, lane16, y)


def kernel(hidden_states, Wg, Wu, Wd, Wsg, Wsu, Wsd, gate_w, shared_gate_w):
    B, S, _ = hidden_states.shape
    x = hidden_states.reshape(B * S, H)
    wg_ext = jnp.concatenate([Wg, Wsg[None]], axis=0)
    wu_ext = jnp.concatenate([Wu, Wsu[None]], axis=0)
    wd_ext = jnp.concatenate([Wd, Wsd[None]], axis=0)
    gwp = jnp.zeros((128, H), jnp.float32).at[:E].set(gate_w).at[E].set(shared_gate_w[0])
    lane16 = jnp.arange(16, dtype=jnp.int32)

    x16 = x.astype(jnp.bfloat16)
    x16_i32 = lax.bitcast_convert_type(x16.reshape(T, H // 2, 2), jnp.int32)
    wg16 = wg_ext.astype(jnp.bfloat16)
    wu16 = wu_ext.astype(jnp.bfloat16)
    wd16 = wd_ext.astype(jnp.bfloat16)

    ids, fvals = _router(x, gwp)
    rank, cnt = _bin_a(ids, lane16)
    pos, bexp = _bin_b(ids, lane16, rank, cnt)
    xs_i32 = _dispatch(x16_i32, pos)
    x_sorted = lax.bitcast_convert_type(
        xs_i32.reshape(P_MAX, H // 2, 1), jnp.bfloat16).reshape(P_MAX, H)
    y = _grouped_mlp(x_sorted, x16, wg16, wu16, wd16, bexp)
    out = _combine(pos, fvals, lane16, y)
    return out.reshape(B, S, H)


# speedup vs baseline: 1.6151x; 1.6151x over previous
import functools

import jax
import jax.numpy as jnp
from jax import lax
from jax.experimental import pallas as pl
from jax.experimental.pallas import tpu as pltpu
from jax.experimental.pallas import tpu_sc as plsc

E = 8
H = 1024
FF = 512
T = 2048
BLK = 128
N_ITEMS = 2 * T
P_MAX = N_ITEMS + E * BLK
G_PAD = P_MAX // BLK
G_TOT = G_PAD + T // BLK
P_TOT = P_MAX + T
NEG = -1e30

NC = 2
NS = 16
NW = NC * NS
IPW = N_ITEMS // NW


def _sigmoid(x):
    return 1.0 / (1.0 + jnp.exp(-x))


def _mesh():
    return plsc.VectorSubcoreMesh(core_axis_name="c", subcore_axis_name="s")


def _wid():
    return lax.axis_index("s") * NC + lax.axis_index("c")


def _shf_init(shf, zero):
    shf[pl.ds(0, 16)] = zero
    shf[pl.ds(32, 16)] = zero


def _shift_up(v, shf, k):
    shf[pl.ds(16, 16)] = v
    return shf[pl.ds(16 - k, 16)]


def _shift_down(v, shf, k):
    shf[pl.ds(16, 16)] = v
    return shf[pl.ds(16 + k, 16)]


def _scan16(v, shf):
    for k in (1, 2, 4, 8):
        v = v + _shift_up(v, shf, k)
    return v


def _propagate0(t, shf):
    for k in (1, 2, 4, 8):
        t = t + _shift_up(t, shf, k)
    return t


def _splat_last(v, shf, lane):
    r = lax.rev(v, (0,))
    return _propagate0(jnp.where(lane == 0, r, jnp.zeros((16,), r.dtype)), shf)


def _splat_at(v, shf, lane, e):
    t = _shift_down(v, shf, e) if e else v
    return _propagate0(jnp.where(lane == 0, t, jnp.zeros((16,), t.dtype)), shf)


def _router_body(x_ref, gwp_ref, i_ref, f_ref):
    lt = lax.dot_general(gwp_ref[...], x_ref[...], (((1,), (1,)), ((), ())),
                         preferred_element_type=jnp.float32)
    row = lax.broadcasted_iota(jnp.int32, lt.shape, 0)
    lm = jnp.where(row < E, lt, NEG)
    m0 = jnp.max(lm, axis=0, keepdims=True)
    a0 = jnp.min(jnp.where(lm == m0, row, 999), axis=0, keepdims=True)
    lm2 = jnp.where(row == a0, NEG, lm)
    m1 = jnp.max(lm2, axis=0, keepdims=True)
    a1 = jnp.min(jnp.where(lm2 == m1, row, 999), axis=0, keepdims=True)
    w0 = _sigmoid(m0 - m1)
    w1 = _sigmoid(m1 - m0)
    sg = _sigmoid(lt[E:E + 1, :])
    r8 = lax.broadcasted_iota(jnp.int32, (8, a0.shape[1]), 0)
    i_ref[...] = jnp.where(r8 == 0, a0, jnp.where(r8 == 1, a1, 0))
    f_ref[...] = jnp.where(r8 == 0, w0,
                           jnp.where(r8 == 1, w1,
                                     jnp.where(r8 == 2, sg, 0.0)))


def _router(x, gwp):
    return pl.pallas_call(
        _router_body,
        grid=(T // BLK,),
        in_specs=[
            pl.BlockSpec((BLK, H), lambda g: (g, 0)),
            pl.BlockSpec((128, H), lambda g: (0, 0)),
        ],
        out_specs=[
            pl.BlockSpec((8, BLK), lambda g: (0, g)),
            pl.BlockSpec((8, BLK), lambda g: (0, g)),
        ],
        out_shape=[
            jax.ShapeDtypeStruct((8, T), jnp.int32),
            jax.ShapeDtypeStruct((8, T), jnp.float32),
        ],
    )(x, gwp)


def _bin_a_kernel(ids_hbm, lane_hbm, rank_hbm, cnt_hbm,
                  eids_v, rank_v, cnt_v, shf_v, lane_v):
    wid = _wid()
    flatbase = wid * IPW
    pltpu.sync_copy(lane_hbm, lane_v)
    lane = lane_v[pl.ds(0, 16)]
    zero = jnp.zeros((16,), jnp.int32)
    _shf_init(shf_v, zero)
    pltpu.sync_copy(ids_hbm.at[flatbase // T, pl.ds(flatbase % T, IPW)], eids_v)
    nv = IPW // 16
    ranks = [zero for _ in range(nv)]
    counts = zero
    for e in range(E):
        run = zero
        for v in range(nv):
            ev = eids_v[pl.ds(v * 16, 16)]
            m = ev == e
            c = _scan16(jnp.where(m, 1, 0), shf_v)
            ranks[v] = jnp.where(m, run + c - 1, ranks[v])
            run = run + _splat_last(c, shf_v, lane)
        counts = jnp.where(lane == e, run, counts)
    for v in range(nv):
        rank_v[pl.ds(v * 16, 16)] = ranks[v]
    cnt_v[pl.ds(0, 16)] = counts
    pltpu.sync_copy(rank_v, rank_hbm.at[pl.ds(flatbase, IPW)])
    pltpu.sync_copy(cnt_v, cnt_hbm.at[pl.ds(wid * 16, 16)])


def _bin_a(ids, lane16):
    f = functools.partial(
        pl.kernel,
        mesh=_mesh(),
        out_type=[
            jax.ShapeDtypeStruct((N_ITEMS,), jnp.int32),
            jax.ShapeDtypeStruct((NW * 16,), jnp.int32),
        ],
        scratch_types=[
            pltpu.VMEM((IPW,), jnp.int32),
            pltpu.VMEM((IPW,), jnp.int32),
            pltpu.VMEM((16,), jnp.int32),
            pltpu.VMEM((48,), jnp.int32),
            pltpu.VMEM((16,), jnp.int32),
        ],
    )(_bin_a_kernel)
    return f(ids, lane16)


def _bin_b_kernel(ids_hbm, lane_hbm, rank_hbm, cnt_hbm, pos_hbm, bexp_hbm,
                  eids_v, rank_v, pos_v, allc_v, shf_v, lane_v, pfx_v, bexp_v):
    wid = _wid()
    flatbase = wid * IPW
    pltpu.sync_copy(lane_hbm, lane_v)
    lane = lane_v[pl.ds(0, 16)]
    zero = jnp.zeros((16,), jnp.int32)
    _shf_init(shf_v, zero)
    pltpu.sync_copy(ids_hbm.at[flatbase // T, pl.ds(flatbase % T, IPW)], eids_v)
    pltpu.sync_copy(rank_hbm.at[pl.ds(flatbase, IPW)], rank_v)
    pltpu.sync_copy(cnt_hbm, allc_v)
    pfx_v[pl.ds(0, 16)] = zero
    total = zero
    for w in range(NW):
        rowv = allc_v[pl.ds(w * 16, 16)]
        total = total + rowv

        @pl.when(w < wid)
        def _acc(rowv=rowv):
            pfx_v[pl.ds(0, 16)] = pfx_v[pl.ds(0, 16)] + rowv

    padded = jnp.bitwise_and(total + (BLK - 1), -BLK)
    off = _scan16(padded, shf_v) - padded
    base = off + pfx_v[pl.ds(0, 16)]
    base_splats = [_splat_at(base, shf_v, lane, e) for e in range(E)]
    nv = IPW // 16
    for v in range(nv):
        ev = eids_v[pl.ds(v * 16, 16)]
        b = zero
        for e in range(E):
            b = jnp.where(ev == e, base_splats[e], b)
        pos_v[pl.ds(v * 16, 16)] = b + rank_v[pl.ds(v * 16, 16)]
    pltpu.sync_copy(pos_v, pos_hbm.at[pl.ds(flatbase, IPW)])

    @pl.when(wid == 0)
    def _sched():
        off_splats = [_splat_at(off, shf_v, lane, e) for e in range(E)]
        for cq in range(4):
            bidx = lane + cq * 16
            bstart = bidx * BLK
            acc = jnp.full((16,), -1, jnp.int32)
            for e in range(E):
                acc = acc + jnp.where(bstart >= off_splats[e], 1, 0)
            acc = jnp.where(bidx >= G_PAD, E, acc)
            bexp_v[pl.ds(cq * 16, 16)] = acc
        pltpu.sync_copy(bexp_v, bexp_hbm)


def _bin_b(ids, lane16, rank, cnt):
    f = functools.partial(
        pl.kernel,
        mesh=_mesh(),
        out_type=[
            jax.ShapeDtypeStruct((N_ITEMS,), jnp.int32),
            jax.ShapeDtypeStruct((64,), jnp.int32),
        ],
        scratch_types=[
            pltpu.VMEM((IPW,), jnp.int32),
            pltpu.VMEM((IPW,), jnp.int32),
            pltpu.VMEM((IPW,), jnp.int32),
            pltpu.VMEM((NW * 16,), jnp.int32),
            pltpu.VMEM((48,), jnp.int32),
            pltpu.VMEM((16,), jnp.int32),
            pltpu.VMEM((16,), jnp.int32),
            pltpu.VMEM((64,), jnp.int32),
        ],
    )(_bin_b_kernel)
    return f(ids, lane16, rank, cnt)


DCHUNK = 32


def _dispatch_kernel(x_hbm, pos_hbm, f_hbm, xs_hbm, wsw_hbm,
                     idx_v, rows_v, wv_v, wsw_v, shf_v, sem):
    wid = _wid()
    flatbase = wid * IPW
    t0 = flatbase % T
    wrow = flatbase // T
    _shf_init(shf_v, jnp.zeros((16,), jnp.float32))
    for cc in range(IPW // DCHUNK):
        pltpu.sync_copy(pos_hbm.at[pl.ds(flatbase + cc * DCHUNK, DCHUNK)], idx_v)
        pltpu.sync_copy(x_hbm.at[pl.ds(t0 + cc * DCHUNK, DCHUNK)], rows_v)
        pltpu.sync_copy(f_hbm.at[wrow, pl.ds(t0 + cc * DCHUNK, DCHUNK)], wv_v)
        for r in range(DCHUNK):
            vi = wv_v[pl.ds((r // 16) * 16, 16)]
            sh = _shift_down(vi, shf_v, r % 16) if r % 16 else vi
            wsw_v[r, pl.ds(0, 16)] = sh
        pltpu.async_copy(rows_v, xs_hbm.at[idx_v], sem).wait()
        pltpu.async_copy(wsw_v, wsw_hbm.at[idx_v], sem).wait()

    @pl.when(wid < NS)
    def _shared():
        for cc in range(IPW // DCHUNK):
            ts = t0 + cc * DCHUNK
            pltpu.sync_copy(f_hbm.at[2, pl.ds(ts, DCHUNK)], wv_v)
            for r in range(DCHUNK):
                vi = wv_v[pl.ds((r // 16) * 16, 16)]
                sh = _shift_down(vi, shf_v, r % 16) if r % 16 else vi
                wsw_v[r, pl.ds(0, 16)] = sh
            pltpu.sync_copy(wsw_v, wsw_hbm.at[pl.ds(P_MAX + ts, DCHUNK)])


def _dispatch(x, pos, fvals):
    f = functools.partial(
        pl.kernel,
        mesh=_mesh(),
        out_type=[
            jax.ShapeDtypeStruct((P_MAX, H), jnp.float32),
            jax.ShapeDtypeStruct((P_TOT, 128), jnp.float32),
        ],
        scratch_types=[
            pltpu.VMEM((DCHUNK,), jnp.int32),
            pltpu.VMEM((DCHUNK, H), jnp.float32),
            pltpu.VMEM((DCHUNK,), jnp.float32),
            pltpu.VMEM((DCHUNK, 128), jnp.float32),
            pltpu.VMEM((48,), jnp.float32),
            pltpu.SemaphoreType.DMA,
        ],
    )(_dispatch_kernel)
    return f(x, pos, fvals)


def _grouped_body(s_ref, xs_ref, x_ref, wg_ref, wu_ref, wd_ref, wsw_ref, y_ref):
    g = pl.program_id(0)
    xb = jnp.where(g < G_PAD, xs_ref[...], x_ref[...])
    hg = lax.dot_general(xb, wg_ref[0], (((1,), (1,)), ((), ())),
                         preferred_element_type=jnp.float32)
    hu = lax.dot_general(xb, wu_ref[0], (((1,), (1,)), ((), ())),
                         preferred_element_type=jnp.float32)
    ha = hg * _sigmoid(hg) * hu
    y = lax.dot_general(ha, wd_ref[0], (((1,), (1,)), ((), ())),
                        preferred_element_type=jnp.float32)
    y_ref[...] = y * wsw_ref[:, 0:1]


def _grouped_mlp(x_sorted, x, wg_ext, wu_ext, wd_ext, bexp, wsw):
    grid_spec = pltpu.PrefetchScalarGridSpec(
        num_scalar_prefetch=1,
        grid=(G_TOT,),
        in_specs=[
            pl.BlockSpec((BLK, H), lambda g, s: (jnp.minimum(g, G_PAD - 1), 0)),
            pl.BlockSpec((BLK, H), lambda g, s: (jnp.maximum(g - G_PAD, 0), 0)),
            pl.BlockSpec((1, FF, H), lambda g, s: (s[g], 0, 0)),
            pl.BlockSpec((1, FF, H), lambda g, s: (s[g], 0, 0)),
            pl.BlockSpec((1, H, FF), lambda g, s: (s[g], 0, 0)),
            pl.BlockSpec((BLK, 128), lambda g, s: (g, 0)),
        ],
        out_specs=pl.BlockSpec((BLK, H), lambda g, s: (g, 0)),
    )
    return pl.pallas_call(
        _grouped_body,
        grid_spec=grid_spec,
        out_shape=jax.ShapeDtypeStruct((P_TOT, H), jnp.float32),
    )(bexp, x_sorted, x, wg_ext, wu_ext, wd_ext, wsw)


TOK_PER_CTILE = T // NW
CCHUNK = 16


def _combine_kernel(pos_hbm, y_hbm, out_hbm,
                    p0_v, p1_v, y0_v, y1_v, ys_v, o_v, sem):
    wid = _wid()

    def body(cc, carry):
        t0 = wid * TOK_PER_CTILE + cc * CCHUNK
        pltpu.sync_copy(pos_hbm.at[pl.ds(t0, CCHUNK)], p0_v)
        pltpu.sync_copy(pos_hbm.at[pl.ds(T + t0, CCHUNK)], p1_v)
        c0 = pltpu.make_async_copy(y_hbm.at[p0_v], y0_v, sem)
        c1 = pltpu.make_async_copy(y_hbm.at[p1_v], y1_v, sem)
        c2 = pltpu.make_async_copy(y_hbm.at[pl.ds(P_MAX + t0, CCHUNK)], ys_v, sem)
        c0.start()
        c1.start()
        c2.start()
        c0.wait()
        c1.wait()
        c2.wait()
        for tt in range(CCHUNK):
            for j in range(H // 16):
                sl = pl.ds(j * 16, 16)
                o_v[tt, sl] = ys_v[tt, sl] + y0_v[tt, sl] + y1_v[tt, sl]
        pltpu.sync_copy(o_v, out_hbm.at[pl.ds(t0, CCHUNK)])
        return carry

    lax.fori_loop(0, TOK_PER_CTILE // CCHUNK, body, 0)


def _combine(pos, y):
    f = functools.partial(
        pl.kernel,
        mesh=_mesh(),
        out_type=jax.ShapeDtypeStruct((T, H), jnp.float32),
        scratch_types=[
            pltpu.VMEM((CCHUNK,), jnp.int32),
            pltpu.VMEM((CCHUNK,), jnp.int32),
            pltpu.VMEM((CCHUNK, H), jnp.float32),
            pltpu.VMEM((CCHUNK, H), jnp.float32),
            pltpu.VMEM((CCHUNK, H), jnp.float32),
            pltpu.VMEM((CCHUNK, H), jnp.float32),
            pltpu.SemaphoreType.DMA,
        ],
    )(_combine_kernel)
    return f(pos, y)


def kernel(hidden_states, Wg, Wu, Wd, Wsg, Wsu, Wsd, gate_w, shared_gate_w):
    B, S, _ = hidden_states.shape
    x = hidden_states.reshape(B * S, H)
    wg_ext = jnp.concatenate([Wg, Wsg[None]], axis=0)
    wu_ext = jnp.concatenate([Wu, Wsu[None]], axis=0)
    wd_ext = jnp.concatenate([Wd, Wsd[None]], axis=0)
    gwp = jnp.zeros((128, H), jnp.float32).at[:E].set(gate_w).at[E].set(shared_gate_w[0])
    lane16 = jnp.arange(16, dtype=jnp.int32)

    ids, fvals = _router(x, gwp)
    rank, cnt = _bin_a(ids, lane16)
    pos, bexp = _bin_b(ids, lane16, rank, cnt)
    x_sorted, wsw = _dispatch(x, pos, fvals)
    y = _grouped_mlp(x_sorted, x, wg_ext, wu_ext, wd_ext, bexp, wsw)
    out = _combine(pos, y)
    return out.reshape(B, S, H)


# speedup vs baseline: 1.8696x; 1.1576x over previous
import functools

import jax
import jax.numpy as jnp
from jax import lax
from jax.experimental import pallas as pl
from jax.experimental.pallas import tpu as pltpu
from jax.experimental.pallas import tpu_sc as plsc

E = 8
H = 1024
FF = 512
T = 2048
BLK = 128
N_ITEMS = 2 * T
P_MAX = N_ITEMS + E * BLK
G_PAD = P_MAX // BLK
G_TOT = G_PAD + T // BLK
P_TOT = P_MAX + T
NEG = -1e30

NC = 2
NS = 16
NW = NC * NS
IPW = N_ITEMS // NW


def _sigmoid(x):
    return 1.0 / (1.0 + jnp.exp(-x))


def _mesh():
    return plsc.VectorSubcoreMesh(core_axis_name="c", subcore_axis_name="s")


def _wid():
    return lax.axis_index("s") * NC + lax.axis_index("c")


def _shf_init(shf, zero):
    shf[pl.ds(0, 16)] = zero
    shf[pl.ds(32, 16)] = zero


def _shift_up(v, shf, k):
    shf[pl.ds(16, 16)] = v
    return shf[pl.ds(16 - k, 16)]


def _shift_down(v, shf, k):
    shf[pl.ds(16, 16)] = v
    return shf[pl.ds(16 + k, 16)]


def _scan16(v, shf):
    for k in (1, 2, 4, 8):
        v = v + _shift_up(v, shf, k)
    return v


def _propagate0(t, shf):
    for k in (1, 2, 4, 8):
        t = t + _shift_up(t, shf, k)
    return t


def _splat_last(v, shf, lane):
    r = lax.rev(v, (0,))
    return _propagate0(jnp.where(lane == 0, r, jnp.zeros((16,), r.dtype)), shf)


def _splat_at(v, shf, lane, e):
    t = _shift_down(v, shf, e) if e else v
    return _propagate0(jnp.where(lane == 0, t, jnp.zeros((16,), t.dtype)), shf)


def _router_body(x_ref, gwp_ref, i_ref, f_ref):
    lt = lax.dot_general(gwp_ref[...], x_ref[...], (((1,), (1,)), ((), ())),
                         preferred_element_type=jnp.float32)
    row = lax.broadcasted_iota(jnp.int32, lt.shape, 0)
    lm = jnp.where(row < E, lt, NEG)
    m0 = jnp.max(lm, axis=0, keepdims=True)
    a0 = jnp.min(jnp.where(lm == m0, row, 999), axis=0, keepdims=True)
    lm2 = jnp.where(row == a0, NEG, lm)
    m1 = jnp.max(lm2, axis=0, keepdims=True)
    a1 = jnp.min(jnp.where(lm2 == m1, row, 999), axis=0, keepdims=True)
    w0 = _sigmoid(m0 - m1)
    w1 = _sigmoid(m1 - m0)
    sg = _sigmoid(lt[E:E + 1, :])
    r8 = lax.broadcasted_iota(jnp.int32, (8, a0.shape[1]), 0)
    i_ref[...] = jnp.where(r8 == 0, a0, jnp.where(r8 == 1, a1, 0))
    f_ref[...] = jnp.where(r8 == 0, w0,
                           jnp.where(r8 == 1, w1,
                                     jnp.where(r8 == 2, sg, 0.0)))


def _router(x, gwp):
    return pl.pallas_call(
        _router_body,
        grid=(T // BLK,),
        in_specs=[
            pl.BlockSpec((BLK, H), lambda g: (g, 0)),
            pl.BlockSpec((128, H), lambda g: (0, 0)),
        ],
        out_specs=[
            pl.BlockSpec((8, BLK), lambda g: (0, g)),
            pl.BlockSpec((8, BLK), lambda g: (0, g)),
        ],
        out_shape=[
            jax.ShapeDtypeStruct((8, T), jnp.int32),
            jax.ShapeDtypeStruct((8, T), jnp.float32),
        ],
    )(x, gwp)


def _bin_a_kernel(ids_hbm, lane_hbm, rank_hbm, cnt_hbm,
                  eids_v, rank_v, cnt_v, shf_v, lane_v):
    wid = _wid()
    flatbase = wid * IPW
    pltpu.sync_copy(lane_hbm, lane_v)
    lane = lane_v[pl.ds(0, 16)]
    zero = jnp.zeros((16,), jnp.int32)
    _shf_init(shf_v, zero)
    pltpu.sync_copy(ids_hbm.at[flatbase // T, pl.ds(flatbase % T, IPW)], eids_v)
    nv = IPW // 16
    ranks = [zero for _ in range(nv)]
    counts = zero
    for e in range(E):
        run = zero
        for v in range(nv):
            ev = eids_v[pl.ds(v * 16, 16)]
            m = ev == e
            c = _scan16(jnp.where(m, 1, 0), shf_v)
            ranks[v] = jnp.where(m, run + c - 1, ranks[v])
            run = run + _splat_last(c, shf_v, lane)
        counts = jnp.where(lane == e, run, counts)
    for v in range(nv):
        rank_v[pl.ds(v * 16, 16)] = ranks[v]
    cnt_v[pl.ds(0, 16)] = counts
    pltpu.sync_copy(rank_v, rank_hbm.at[pl.ds(flatbase, IPW)])
    pltpu.sync_copy(cnt_v, cnt_hbm.at[pl.ds(wid * 16, 16)])


def _bin_a(ids, lane16):
    f = functools.partial(
        pl.kernel,
        mesh=_mesh(),
        out_type=[
            jax.ShapeDtypeStruct((N_ITEMS,), jnp.int32),
            jax.ShapeDtypeStruct((NW * 16,), jnp.int32),
        ],
        scratch_types=[
            pltpu.VMEM((IPW,), jnp.int32),
            pltpu.VMEM((IPW,), jnp.int32),
            pltpu.VMEM((16,), jnp.int32),
            pltpu.VMEM((48,), jnp.int32),
            pltpu.VMEM((16,), jnp.int32),
        ],
    )(_bin_a_kernel)
    return f(ids, lane16)


def _bin_b_kernel(ids_hbm, lane_hbm, rank_hbm, cnt_hbm, pos_hbm, bexp_hbm,
                  eids_v, rank_v, pos_v, allc_v, shf_v, lane_v, pfx_v, bexp_v):
    wid = _wid()
    flatbase = wid * IPW
    pltpu.sync_copy(lane_hbm, lane_v)
    lane = lane_v[pl.ds(0, 16)]
    zero = jnp.zeros((16,), jnp.int32)
    _shf_init(shf_v, zero)
    pltpu.sync_copy(ids_hbm.at[flatbase // T, pl.ds(flatbase % T, IPW)], eids_v)
    pltpu.sync_copy(rank_hbm.at[pl.ds(flatbase, IPW)], rank_v)
    pltpu.sync_copy(cnt_hbm, allc_v)
    pfx_v[pl.ds(0, 16)] = zero
    total = zero
    for w in range(NW):
        rowv = allc_v[pl.ds(w * 16, 16)]
        total = total + rowv

        @pl.when(w < wid)
        def _acc(rowv=rowv):
            pfx_v[pl.ds(0, 16)] = pfx_v[pl.ds(0, 16)] + rowv

    padded = jnp.bitwise_and(total + (BLK - 1), -BLK)
    off = _scan16(padded, shf_v) - padded
    base = off + pfx_v[pl.ds(0, 16)]
    base_splats = [_splat_at(base, shf_v, lane, e) for e in range(E)]
    nv = IPW // 16
    for v in range(nv):
        ev = eids_v[pl.ds(v * 16, 16)]
        b = zero
        for e in range(E):
            b = jnp.where(ev == e, base_splats[e], b)
        pos_v[pl.ds(v * 16, 16)] = b + rank_v[pl.ds(v * 16, 16)]
    pltpu.sync_copy(pos_v, pos_hbm.at[pl.ds(flatbase, IPW)])

    @pl.when(wid == 0)
    def _sched():
        off_splats = [_splat_at(off, shf_v, lane, e) for e in range(E)]
        for cq in range(4):
            bidx = lane + cq * 16
            bstart = bidx * BLK
            acc = jnp.full((16,), -1, jnp.int32)
            for e in range(E):
                acc = acc + jnp.where(bstart >= off_splats[e], 1, 0)
            acc = jnp.where(bidx >= G_PAD, E, acc)
            bexp_v[pl.ds(cq * 16, 16)] = acc
        pltpu.sync_copy(bexp_v, bexp_hbm)


def _bin_b(ids, lane16, rank, cnt):
    f = functools.partial(
        pl.kernel,
        mesh=_mesh(),
        out_type=[
            jax.ShapeDtypeStruct((N_ITEMS,), jnp.int32),
            jax.ShapeDtypeStruct((64,), jnp.int32),
        ],
        scratch_types=[
            pltpu.VMEM((IPW,), jnp.int32),
            pltpu.VMEM((IPW,), jnp.int32),
            pltpu.VMEM((IPW,), jnp.int32),
            pltpu.VMEM((NW * 16,), jnp.int32),
            pltpu.VMEM((48,), jnp.int32),
            pltpu.VMEM((16,), jnp.int32),
            pltpu.VMEM((16,), jnp.int32),
            pltpu.VMEM((64,), jnp.int32),
        ],
    )(_bin_b_kernel)
    return f(ids, lane16, rank, cnt)


DCHUNK = 32
NCHUNKS = IPW // DCHUNK


def _dispatch_kernel(x_hbm, pos_hbm, f_hbm, xs_hbm, wsw_hbm,
                     idx_v, rows_v, wv_v, wsw_v, shf_v, semi, sems):
    wid = _wid()
    flatbase = wid * IPW
    t0 = flatbase % T
    wrow = flatbase // T
    _shf_init(shf_v, jnp.zeros((16,), jnp.float32))
    for cc in range(NCHUNKS):
        b = cc % 2
        c1 = pltpu.make_async_copy(
            pos_hbm.at[pl.ds(flatbase + cc * DCHUNK, DCHUNK)], idx_v.at[b], semi)
        c2 = pltpu.make_async_copy(
            x_hbm.at[pl.ds(t0 + cc * DCHUNK, DCHUNK)], rows_v.at[b], semi)
        c3 = pltpu.make_async_copy(
            f_hbm.at[wrow, pl.ds(t0 + cc * DCHUNK, DCHUNK)], wv_v.at[b], semi)
        c1.start()
        c2.start()
        c3.start()
        c1.wait()
        c2.wait()
        c3.wait()
        for r in range(DCHUNK):
            vi = wv_v[b, pl.ds((r // 16) * 16, 16)]
            sh = _shift_down(vi, shf_v, r % 16) if r % 16 else vi
            wsw_v[b, r, pl.ds(0, 16)] = sh
        s1 = pltpu.make_async_copy(rows_v.at[b], xs_hbm.at[idx_v.at[b]], sems)
        s2 = pltpu.make_async_copy(wsw_v.at[b], wsw_hbm.at[idx_v.at[b]], sems)
        s1.start()
        s2.start()
        s1.wait()
        s2.wait()


def _dispatch(x, pos, fvals):
    f = functools.partial(
        pl.kernel,
        mesh=_mesh(),
        out_type=[
            jax.ShapeDtypeStruct((P_MAX, H), jnp.float32),
            jax.ShapeDtypeStruct((P_MAX, 128), jnp.float32),
        ],
        scratch_types=[
            pltpu.VMEM((2, DCHUNK), jnp.int32),
            pltpu.VMEM((2, DCHUNK, H), jnp.float32),
            pltpu.VMEM((2, DCHUNK), jnp.float32),
            pltpu.VMEM((2, DCHUNK, 128), jnp.float32),
            pltpu.VMEM((48,), jnp.float32),
            pltpu.SemaphoreType.DMA,
            pltpu.SemaphoreType.DMA,
        ],
    )(_dispatch_kernel)
    return f(x, pos, fvals)


def _shared_body(x_ref, wg_ref, wu_ref, wd_ref, y_ref):
    xb = x_ref[...]
    hg = lax.dot_general(xb, wg_ref[...], (((1,), (1,)), ((), ())),
                         preferred_element_type=jnp.float32)
    hu = lax.dot_general(xb, wu_ref[...], (((1,), (1,)), ((), ())),
                         preferred_element_type=jnp.float32)
    ha = hg * _sigmoid(hg) * hu
    y_ref[...] = lax.dot_general(ha, wd_ref[...], (((1,), (1,)), ((), ())),
                                 preferred_element_type=jnp.float32)


def _mlp_shared(x, wsg, wsu, wsd):
    return pl.pallas_call(
        _shared_body,
        grid=(T // BLK,),
        in_specs=[
            pl.BlockSpec((BLK, H), lambda g: (g, 0)),
            pl.BlockSpec((FF, H), lambda g: (0, 0)),
            pl.BlockSpec((FF, H), lambda g: (0, 0)),
            pl.BlockSpec((H, FF), lambda g: (0, 0)),
        ],
        out_specs=pl.BlockSpec((BLK, H), lambda g: (g, 0)),
        out_shape=jax.ShapeDtypeStruct((T, H), jnp.float32),
    )(x, wsg, wsu, wsd)


def _routed_body(s_ref, xs_ref, wg_ref, wu_ref, wd_ref, wsw_ref, y_ref):
    xb = xs_ref[...]
    hg = lax.dot_general(xb, wg_ref[0], (((1,), (1,)), ((), ())),
                         preferred_element_type=jnp.float32)
    hu = lax.dot_general(xb, wu_ref[0], (((1,), (1,)), ((), ())),
                         preferred_element_type=jnp.float32)
    ha = hg * _sigmoid(hg) * hu
    y = lax.dot_general(ha, wd_ref[0], (((1,), (1,)), ((), ())),
                        preferred_element_type=jnp.float32)
    y_ref[...] = y * wsw_ref[:, 0:1]


def _mlp_routed(x_sorted, wg, wu, wd, bexp, wsw):
    grid_spec = pltpu.PrefetchScalarGridSpec(
        num_scalar_prefetch=1,
        grid=(G_PAD,),
        in_specs=[
            pl.BlockSpec((BLK, H), lambda g, s: (g, 0)),
            pl.BlockSpec((1, FF, H), lambda g, s: (s[g], 0, 0)),
            pl.BlockSpec((1, FF, H), lambda g, s: (s[g], 0, 0)),
            pl.BlockSpec((1, H, FF), lambda g, s: (s[g], 0, 0)),
            pl.BlockSpec((BLK, 128), lambda g, s: (g, 0)),
        ],
        out_specs=pl.BlockSpec((BLK, H), lambda g, s: (g, 0)),
    )
    return pl.pallas_call(
        _routed_body,
        grid_spec=grid_spec,
        out_shape=jax.ShapeDtypeStruct((P_MAX, H), jnp.float32),
    )(bexp, x_sorted, wg, wu, wd, wsw)


TOK_PER_CTILE = T // NW
CCHUNK = 8
NCC = TOK_PER_CTILE // CCHUNK


def _combine_kernel(pos_hbm, f_hbm, lane_hbm, yr_hbm, ys_hbm, out_hbm,
                    p0_v, p1_v, sg_v, y0_v, y1_v, ys_v, o_v, shf_v, lane_v,
                    sem_a, sem_b):
    wid = _wid()
    pltpu.sync_copy(lane_hbm, lane_v)
    lane = lane_v[pl.ds(0, 16)]
    _shf_init(shf_v, jnp.zeros((16,), jnp.float32))
    tbase = wid * TOK_PER_CTILE

    def body(cc, carry):
        t0 = tbase + cc * CCHUNK
        pltpu.sync_copy(pos_hbm.at[pl.ds(t0, CCHUNK)], p0_v.at[0])
        pltpu.sync_copy(pos_hbm.at[pl.ds(T + t0, CCHUNK)], p1_v.at[0])
        pltpu.sync_copy(f_hbm.at[2, pl.ds(t0, CCHUNK)], sg_v.at[0])
        g0 = pltpu.make_async_copy(yr_hbm.at[p0_v.at[0]],
                                   y0_v.at[pl.ds(0, CCHUNK)], sem_a)
        g1 = pltpu.make_async_copy(yr_hbm.at[p1_v.at[0]],
                                   y1_v.at[pl.ds(0, CCHUNK)], sem_a)
        g2 = pltpu.make_async_copy(ys_hbm.at[pl.ds(t0, CCHUNK)],
                                   ys_v.at[pl.ds(0, CCHUNK)], sem_a)
        g0.start()
        g1.start()
        g2.start()
        g0.wait()
        g1.wait()
        g2.wait()
        sga = sg_v[0, pl.ds(0, 16)]
        for tt in range(CCHUNK):
            sgs = _splat_at(sga, shf_v, lane, tt)
            for j in range(H // 16):
                sl = pl.ds(j * 16, 16)
                o_v[tt, sl] = (sgs * ys_v[tt, sl] + y0_v[tt, sl]
                               + y1_v[tt, sl])
        pltpu.sync_copy(o_v, out_hbm.at[pl.ds(t0, CCHUNK)])
        return carry

    lax.fori_loop(0, NCC, body, 0)


def _combine(pos, fvals, lane16, y_r, y_s):
    f = functools.partial(
        pl.kernel,
        mesh=_mesh(),
        out_type=jax.ShapeDtypeStruct((T, H), jnp.float32),
        scratch_types=[
            pltpu.VMEM((2, CCHUNK), jnp.int32),
            pltpu.VMEM((2, CCHUNK), jnp.int32),
            pltpu.VMEM((2, CCHUNK), jnp.float32),
            pltpu.VMEM((2 * CCHUNK, H), jnp.float32),
            pltpu.VMEM((2 * CCHUNK, H), jnp.float32),
            pltpu.VMEM((2 * CCHUNK, H), jnp.float32),
            pltpu.VMEM((CCHUNK, H), jnp.float32),
            pltpu.VMEM((48,), jnp.float32),
            pltpu.VMEM((16,), jnp.int32),
            pltpu.SemaphoreType.DMA,
            pltpu.SemaphoreType.DMA,
        ],
    )(_combine_kernel)
    return f(pos, fvals, lane16, y_r, y_s)


def kernel(hidden_states, Wg, Wu, Wd, Wsg, Wsu, Wsd, gate_w, shared_gate_w):
    B, S, _ = hidden_states.shape
    x = hidden_states.reshape(B * S, H)
    gwp = jnp.zeros((128, H), jnp.float32).at[:E].set(gate_w).at[E].set(shared_gate_w[0])
    lane16 = jnp.arange(16, dtype=jnp.int32)

    ids, fvals = _router(x, gwp)
    y_s = _mlp_shared(x, Wsg, Wsu, Wsd)
    rank, cnt = _bin_a(ids, lane16)
    pos, bexp = _bin_b(ids, lane16, rank, cnt)
    x_sorted, wsw = _dispatch(x, pos, fvals)
    y_r = _mlp_routed(x_sorted, Wg, Wu, Wd, bexp, wsw)
    out = _combine(pos, fvals, lane16, y_r, y_s)
    return out.reshape(B, S, H)


# speedup vs baseline: 1.9359x; 1.0354x over previous
import functools

import jax
import jax.numpy as jnp
from jax import lax
from jax.experimental import pallas as pl
from jax.experimental.pallas import tpu as pltpu
from jax.experimental.pallas import tpu_sc as plsc

E = 8
H = 1024
FF = 512
T = 2048
BLK = 128
N_ITEMS = 2 * T
P_MAX = N_ITEMS + E * BLK
G_PAD = P_MAX // BLK
G_TOT = G_PAD + T // BLK
P_TOT = P_MAX + T
NEG = -1e30

NC = 2
NS = 16
NW = NC * NS
IPW = N_ITEMS // NW


def _sigmoid(x):
    return 1.0 / (1.0 + jnp.exp(-x))


def _mesh():
    return plsc.VectorSubcoreMesh(core_axis_name="c", subcore_axis_name="s")


def _wid():
    return lax.axis_index("s") * NC + lax.axis_index("c")


def _shf_init(shf, zero):
    shf[pl.ds(0, 16)] = zero
    shf[pl.ds(32, 16)] = zero


def _shift_up(v, shf, k):
    shf[pl.ds(16, 16)] = v
    return shf[pl.ds(16 - k, 16)]


def _shift_down(v, shf, k):
    shf[pl.ds(16, 16)] = v
    return shf[pl.ds(16 + k, 16)]


def _scan16(v, shf):
    for k in (1, 2, 4, 8):
        v = v + _shift_up(v, shf, k)
    return v


def _propagate0(t, shf):
    for k in (1, 2, 4, 8):
        t = t + _shift_up(t, shf, k)
    return t


def _splat_last(v, shf, lane):
    r = lax.rev(v, (0,))
    return _propagate0(jnp.where(lane == 0, r, jnp.zeros((16,), r.dtype)), shf)


def _splat_at(v, shf, lane, e):
    t = _shift_down(v, shf, e) if e else v
    return _propagate0(jnp.where(lane == 0, t, jnp.zeros((16,), t.dtype)), shf)


def _router_body(x_ref, gwp_ref, i_ref, f_ref):
    lt = lax.dot_general(gwp_ref[...], x_ref[...], (((1,), (1,)), ((), ())),
                         preferred_element_type=jnp.float32)
    row = lax.broadcasted_iota(jnp.int32, lt.shape, 0)
    lm = jnp.where(row < E, lt, NEG)
    m0 = jnp.max(lm, axis=0, keepdims=True)
    a0 = jnp.min(jnp.where(lm == m0, row, 999), axis=0, keepdims=True)
    lm2 = jnp.where(row == a0, NEG, lm)
    m1 = jnp.max(lm2, axis=0, keepdims=True)
    a1 = jnp.min(jnp.where(lm2 == m1, row, 999), axis=0, keepdims=True)
    w0 = _sigmoid(m0 - m1)
    w1 = _sigmoid(m1 - m0)
    sg = _sigmoid(lt[E:E + 1, :])
    r8 = lax.broadcasted_iota(jnp.int32, (8, a0.shape[1]), 0)
    i_ref[...] = jnp.where(r8 == 0, a0, jnp.where(r8 == 1, a1, 0))
    f_ref[...] = jnp.where(r8 == 0, w0,
                           jnp.where(r8 == 1, w1,
                                     jnp.where(r8 == 2, sg, 0.0)))


def _router(x, gwp):
    return pl.pallas_call(
        _router_body,
        grid=(T // BLK,),
        in_specs=[
            pl.BlockSpec((BLK, H), lambda g: (g, 0)),
            pl.BlockSpec((128, H), lambda g: (0, 0)),
        ],
        out_specs=[
            pl.BlockSpec((8, BLK), lambda g: (0, g)),
            pl.BlockSpec((8, BLK), lambda g: (0, g)),
        ],
        out_shape=[
            jax.ShapeDtypeStruct((8, T), jnp.int32),
            jax.ShapeDtypeStruct((8, T), jnp.float32),
        ],
    )(x, gwp)


def _bin_a_kernel(ids_hbm, lane_hbm, rank_hbm, cnt_hbm,
                  eids_v, rank_v, cnt_v, shf_v, lane_v):
    wid = _wid()
    flatbase = wid * IPW
    pltpu.sync_copy(lane_hbm, lane_v)
    lane = lane_v[pl.ds(0, 16)]
    zero = jnp.zeros((16,), jnp.int32)
    _shf_init(shf_v, zero)
    pltpu.sync_copy(ids_hbm.at[flatbase // T, pl.ds(flatbase % T, IPW)], eids_v)
    nv = IPW // 16
    ranks = [zero for _ in range(nv)]
    counts = zero
    for e in range(E):
        run = zero
        for v in range(nv):
            ev = eids_v[pl.ds(v * 16, 16)]
            m = ev == e
            c = _scan16(jnp.where(m, 1, 0), shf_v)
            ranks[v] = jnp.where(m, run + c - 1, ranks[v])
            run = run + _splat_last(c, shf_v, lane)
        counts = jnp.where(lane == e, run, counts)
    for v in range(nv):
        rank_v[pl.ds(v * 16, 16)] = ranks[v]
    cnt_v[pl.ds(0, 16)] = counts
    pltpu.sync_copy(rank_v, rank_hbm.at[pl.ds(flatbase, IPW)])
    pltpu.sync_copy(cnt_v, cnt_hbm.at[pl.ds(wid * 16, 16)])


def _bin_a(ids, lane16):
    f = functools.partial(
        pl.kernel,
        mesh=_mesh(),
        out_type=[
            jax.ShapeDtypeStruct((N_ITEMS,), jnp.int32),
            jax.ShapeDtypeStruct((NW * 16,), jnp.int32),
        ],
        scratch_types=[
            pltpu.VMEM((IPW,), jnp.int32),
            pltpu.VMEM((IPW,), jnp.int32),
            pltpu.VMEM((16,), jnp.int32),
            pltpu.VMEM((48,), jnp.int32),
            pltpu.VMEM((16,), jnp.int32),
        ],
    )(_bin_a_kernel)
    return f(ids, lane16)


def _bin_b_kernel(ids_hbm, lane_hbm, rank_hbm, cnt_hbm, pos_hbm, bexp_hbm,
                  eids_v, rank_v, pos_v, allc_v, shf_v, lane_v, pfx_v, bexp_v):
    wid = _wid()
    flatbase = wid * IPW
    pltpu.sync_copy(lane_hbm, lane_v)
    lane = lane_v[pl.ds(0, 16)]
    zero = jnp.zeros((16,), jnp.int32)
    _shf_init(shf_v, zero)
    pltpu.sync_copy(ids_hbm.at[flatbase // T, pl.ds(flatbase % T, IPW)], eids_v)
    pltpu.sync_copy(rank_hbm.at[pl.ds(flatbase, IPW)], rank_v)
    pltpu.sync_copy(cnt_hbm, allc_v)
    pfx_v[pl.ds(0, 16)] = zero
    total = zero
    for w in range(NW):
        rowv = allc_v[pl.ds(w * 16, 16)]
        total = total + rowv

        @pl.when(w < wid)
        def _acc(rowv=rowv):
            pfx_v[pl.ds(0, 16)] = pfx_v[pl.ds(0, 16)] + rowv

    padded = jnp.bitwise_and(total + (BLK - 1), -BLK)
    off = _scan16(padded, shf_v) - padded
    base = off + pfx_v[pl.ds(0, 16)]
    base_splats = [_splat_at(base, shf_v, lane, e) for e in range(E)]
    nv = IPW // 16
    for v in range(nv):
        ev = eids_v[pl.ds(v * 16, 16)]
        b = zero
        for e in range(E):
            b = jnp.where(ev == e, base_splats[e], b)
        pos_v[pl.ds(v * 16, 16)] = b + rank_v[pl.ds(v * 16, 16)]
    pltpu.sync_copy(pos_v, pos_hbm.at[pl.ds(flatbase, IPW)])

    @pl.when(wid == 0)
    def _sched():
        off_splats = [_splat_at(off, shf_v, lane, e) for e in range(E)]
        for cq in range(4):
            bidx = lane + cq * 16
            bstart = bidx * BLK
            acc = jnp.full((16,), -1, jnp.int32)
            for e in range(E):
                acc = acc + jnp.where(bstart >= off_splats[e], 1, 0)
            acc = jnp.where(bidx >= G_PAD, E, acc)
            bexp_v[pl.ds(cq * 16, 16)] = acc
        pltpu.sync_copy(bexp_v, bexp_hbm)


def _bin_b(ids, lane16, rank, cnt):
    f = functools.partial(
        pl.kernel,
        mesh=_mesh(),
        out_type=[
            jax.ShapeDtypeStruct((N_ITEMS,), jnp.int32),
            jax.ShapeDtypeStruct((64,), jnp.int32),
        ],
        scratch_types=[
            pltpu.VMEM((IPW,), jnp.int32),
            pltpu.VMEM((IPW,), jnp.int32),
            pltpu.VMEM((IPW,), jnp.int32),
            pltpu.VMEM((NW * 16,), jnp.int32),
            pltpu.VMEM((48,), jnp.int32),
            pltpu.VMEM((16,), jnp.int32),
            pltpu.VMEM((16,), jnp.int32),
            pltpu.VMEM((64,), jnp.int32),
        ],
    )(_bin_b_kernel)
    return f(ids, lane16, rank, cnt)


DCHUNK = 32
NCHUNKS = IPW // DCHUNK


def _dispatch_kernel(x_hbm, pos_hbm, f_hbm, xs_hbm, wsw_hbm,
                     idx_v, rows_v, wv_v, wsw_v, shf_v, semi, sems):
    wid = _wid()
    flatbase = wid * IPW
    t0 = flatbase % T
    wrow = flatbase // T
    _shf_init(shf_v, jnp.zeros((16,), jnp.float32))
    for cc in range(NCHUNKS):
        b = cc % 2
        c1 = pltpu.make_async_copy(
            pos_hbm.at[pl.ds(flatbase + cc * DCHUNK, DCHUNK)], idx_v.at[b], semi)
        c2 = pltpu.make_async_copy(
            x_hbm.at[pl.ds(t0 + cc * DCHUNK, DCHUNK)], rows_v.at[b], semi)
        c3 = pltpu.make_async_copy(
            f_hbm.at[wrow, pl.ds(t0 + cc * DCHUNK, DCHUNK)], wv_v.at[b], semi)
        c1.start()
        c2.start()
        c3.start()
        c1.wait()
        c2.wait()
        c3.wait()
        for r in range(DCHUNK):
            vi = wv_v[b, pl.ds((r // 16) * 16, 16)]
            sh = _shift_down(vi, shf_v, r % 16) if r % 16 else vi
            wsw_v[b, r, pl.ds(0, 16)] = sh
        s1 = pltpu.make_async_copy(rows_v.at[b], xs_hbm.at[idx_v.at[b]], sems)
        s2 = pltpu.make_async_copy(wsw_v.at[b], wsw_hbm.at[idx_v.at[b]], sems)
        s1.start()
        s2.start()
        s1.wait()
        s2.wait()


def _dispatch(x, pos, fvals):
    f = functools.partial(
        pl.kernel,
        mesh=_mesh(),
        out_type=[
            jax.ShapeDtypeStruct((P_MAX, H), jnp.float32),
            jax.ShapeDtypeStruct((P_MAX, 128), jnp.float32),
        ],
        scratch_types=[
            pltpu.VMEM((2, DCHUNK), jnp.int32),
            pltpu.VMEM((2, DCHUNK, H), jnp.float32),
            pltpu.VMEM((2, DCHUNK), jnp.float32),
            pltpu.VMEM((2, DCHUNK, 128), jnp.float32),
            pltpu.VMEM((48,), jnp.float32),
            pltpu.SemaphoreType.DMA,
            pltpu.SemaphoreType.DMA,
        ],
    )(_dispatch_kernel)
    return f(x, pos, fvals)


def _shared_body(x_ref, wg_ref, wu_ref, wd_ref, sgw_ref, y_ref):
    xb = x_ref[...]
    hg = lax.dot_general(xb, wg_ref[...], (((1,), (1,)), ((), ())),
                         preferred_element_type=jnp.float32)
    hu = lax.dot_general(xb, wu_ref[...], (((1,), (1,)), ((), ())),
                         preferred_element_type=jnp.float32)
    ha = hg * _sigmoid(hg) * hu
    y = lax.dot_general(ha, wd_ref[...], (((1,), (1,)), ((), ())),
                        preferred_element_type=jnp.float32)
    gate = _sigmoid(lax.dot_general(xb, sgw_ref[...], (((1,), (1,)), ((), ())),
                                    preferred_element_type=jnp.float32))
    y_ref[...] = y * gate[:, 0:1]


def _mlp_shared(x, wsg, wsu, wsd, sgwp):
    return pl.pallas_call(
        _shared_body,
        grid=(T // BLK,),
        in_specs=[
            pl.BlockSpec((BLK, H), lambda g: (g, 0)),
            pl.BlockSpec((FF, H), lambda g: (0, 0)),
            pl.BlockSpec((FF, H), lambda g: (0, 0)),
            pl.BlockSpec((H, FF), lambda g: (0, 0)),
            pl.BlockSpec((128, H), lambda g: (0, 0)),
        ],
        out_specs=pl.BlockSpec((BLK, H), lambda g: (g, 0)),
        out_shape=jax.ShapeDtypeStruct((T, H), jnp.float32),
    )(x, wsg, wsu, wsd, sgwp)


def _routed_body(s_ref, xs_ref, wg_ref, wu_ref, wd_ref, wsw_ref, y_ref):
    xb = xs_ref[...]
    hg = lax.dot_general(xb, wg_ref[0], (((1,), (1,)), ((), ())),
                         preferred_element_type=jnp.float32)
    hu = lax.dot_general(xb, wu_ref[0], (((1,), (1,)), ((), ())),
                         preferred_element_type=jnp.float32)
    ha = hg * _sigmoid(hg) * hu
    y = lax.dot_general(ha, wd_ref[0], (((1,), (1,)), ((), ())),
                        preferred_element_type=jnp.float32)
    y_ref[...] = y * wsw_ref[:, 0:1]


def _mlp_routed(x_sorted, wg, wu, wd, bexp, wsw):
    grid_spec = pltpu.PrefetchScalarGridSpec(
        num_scalar_prefetch=1,
        grid=(G_PAD,),
        in_specs=[
            pl.BlockSpec((BLK, H), lambda g, s: (g, 0)),
            pl.BlockSpec((1, FF, H), lambda g, s: (s[g], 0, 0)),
            pl.BlockSpec((1, FF, H), lambda g, s: (s[g], 0, 0)),
            pl.BlockSpec((1, H, FF), lambda g, s: (s[g], 0, 0)),
            pl.BlockSpec((BLK, 128), lambda g, s: (g, 0)),
        ],
        out_specs=pl.BlockSpec((BLK, H), lambda g, s: (g, 0)),
    )
    return pl.pallas_call(
        _routed_body,
        grid_spec=grid_spec,
        out_shape=jax.ShapeDtypeStruct((P_MAX, H), jnp.float32),
    )(bexp, x_sorted, wg, wu, wd, wsw)


TOK_PER_CTILE = T // NW
CCHUNK = 16
NCC = TOK_PER_CTILE // CCHUNK


def _combine_kernel(pos_hbm, yr_hbm, ys_hbm, out_hbm,
                    p0_v, p1_v, y0_v, y1_v, ys_v, o_v, sem_a):
    wid = _wid()
    tbase = wid * TOK_PER_CTILE

    def body(cc, carry):
        t0 = tbase + cc * CCHUNK
        pltpu.sync_copy(pos_hbm.at[pl.ds(t0, CCHUNK)], p0_v)
        pltpu.sync_copy(pos_hbm.at[pl.ds(T + t0, CCHUNK)], p1_v)
        g0 = pltpu.make_async_copy(yr_hbm.at[p0_v], y0_v, sem_a)
        g1 = pltpu.make_async_copy(yr_hbm.at[p1_v], y1_v, sem_a)
        g2 = pltpu.make_async_copy(ys_hbm.at[pl.ds(t0, CCHUNK)], ys_v, sem_a)
        g0.start()
        g1.start()
        g2.start()
        g0.wait()
        g1.wait()
        g2.wait()
        for tt in range(CCHUNK):
            for j in range(H // 16):
                sl = pl.ds(j * 16, 16)
                o_v[tt, sl] = ys_v[tt, sl] + y0_v[tt, sl] + y1_v[tt, sl]
        pltpu.sync_copy(o_v, out_hbm.at[pl.ds(t0, CCHUNK)])
        return carry

    lax.fori_loop(0, NCC, body, 0)


def _combine(pos, y_r, y_s):
    f = functools.partial(
        pl.kernel,
        mesh=_mesh(),
        out_type=jax.ShapeDtypeStruct((T, H), jnp.float32),
        scratch_types=[
            pltpu.VMEM((CCHUNK,), jnp.int32),
            pltpu.VMEM((CCHUNK,), jnp.int32),
            pltpu.VMEM((CCHUNK, H), jnp.float32),
            pltpu.VMEM((CCHUNK, H), jnp.float32),
            pltpu.VMEM((CCHUNK, H), jnp.float32),
            pltpu.VMEM((CCHUNK, H), jnp.float32),
            pltpu.SemaphoreType.DMA,
        ],
    )(_combine_kernel)
    return f(pos, y_r, y_s)


def kernel(hidden_states, Wg, Wu, Wd, Wsg, Wsu, Wsd, gate_w, shared_gate_w):
    B, S, _ = hidden_states.shape
    x = hidden_states.reshape(B * S, H)
    gwp = jnp.zeros((128, H), jnp.float32).at[:E].set(gate_w).at[E].set(shared_gate_w[0])
    sgwp = jnp.zeros((128, H), jnp.float32).at[0].set(shared_gate_w[0])
    lane16 = jnp.arange(16, dtype=jnp.int32)

    ids, fvals = _router(x, gwp)
    y_s = _mlp_shared(x, Wsg, Wsu, Wsd, sgwp)
    rank, cnt = _bin_a(ids, lane16)
    pos, bexp = _bin_b(ids, lane16, rank, cnt)
    x_sorted, wsw = _dispatch(x, pos, fvals)
    y_r = _mlp_routed(x_sorted, Wg, Wu, Wd, bexp, wsw)
    out = _combine(pos, y_r, y_s)
    return out.reshape(B, S, H)


# speedup vs baseline: 1.9502x; 1.0074x over previous
import functools

import jax
import jax.numpy as jnp
from jax import lax
from jax.experimental import pallas as pl
from jax.experimental.pallas import tpu as pltpu
from jax.experimental.pallas import tpu_sc as plsc

E = 8
H = 1024
FF = 512
T = 2048
BLK = 128
N_ITEMS = 2 * T
P_MAX = N_ITEMS + E * BLK
G_PAD = P_MAX // BLK
G_TOT = G_PAD + T // BLK
P_TOT = P_MAX + T
NEG = -1e30

NC = 2
NS = 16
NW = NC * NS
IPW = N_ITEMS // NW


def _sigmoid(x):
    return 1.0 / (1.0 + jnp.exp(-x))


def _mesh():
    return plsc.VectorSubcoreMesh(core_axis_name="c", subcore_axis_name="s")


def _wid():
    return lax.axis_index("s") * NC + lax.axis_index("c")


def _shf_init(shf, zero):
    shf[pl.ds(0, 16)] = zero
    shf[pl.ds(32, 16)] = zero


def _shift_up(v, shf, k):
    shf[pl.ds(16, 16)] = v
    return shf[pl.ds(16 - k, 16)]


def _shift_down(v, shf, k):
    shf[pl.ds(16, 16)] = v
    return shf[pl.ds(16 + k, 16)]


def _scan16(v, shf):
    for k in (1, 2, 4, 8):
        v = v + _shift_up(v, shf, k)
    return v


def _propagate0(t, shf):
    for k in (1, 2, 4, 8):
        t = t + _shift_up(t, shf, k)
    return t


def _splat_last(v, shf, lane):
    r = lax.rev(v, (0,))
    return _propagate0(jnp.where(lane == 0, r, jnp.zeros((16,), r.dtype)), shf)


def _splat_at(v, shf, lane, e):
    t = _shift_down(v, shf, e) if e else v
    return _propagate0(jnp.where(lane == 0, t, jnp.zeros((16,), t.dtype)), shf)


def _router_body(x_ref, gwp_ref, i_ref, f_ref):
    lt = lax.dot_general(gwp_ref[...], x_ref[...], (((1,), (1,)), ((), ())),
                         preferred_element_type=jnp.float32)
    row = lax.broadcasted_iota(jnp.int32, lt.shape, 0)
    lm = jnp.where(row < E, lt, NEG)
    m0 = jnp.max(lm, axis=0, keepdims=True)
    a0 = jnp.min(jnp.where(lm == m0, row, 999), axis=0, keepdims=True)
    lm2 = jnp.where(row == a0, NEG, lm)
    m1 = jnp.max(lm2, axis=0, keepdims=True)
    a1 = jnp.min(jnp.where(lm2 == m1, row, 999), axis=0, keepdims=True)
    w0 = _sigmoid(m0 - m1)
    w1 = _sigmoid(m1 - m0)
    sg = _sigmoid(lt[E:E + 1, :])
    r8 = lax.broadcasted_iota(jnp.int32, (8, a0.shape[1]), 0)
    i_ref[...] = jnp.where(r8 == 0, a0, jnp.where(r8 == 1, a1, 0))
    f_ref[...] = jnp.where(r8 == 0, w0,
                           jnp.where(r8 == 1, w1,
                                     jnp.where(r8 == 2, sg, 0.0)))


def _router(x, gwp):
    return pl.pallas_call(
        _router_body,
        grid=(T // BLK,),
        in_specs=[
            pl.BlockSpec((BLK, H), lambda g: (g, 0)),
            pl.BlockSpec((128, H), lambda g: (0, 0)),
        ],
        out_specs=[
            pl.BlockSpec((8, BLK), lambda g: (0, g)),
            pl.BlockSpec((8, BLK), lambda g: (0, g)),
        ],
        out_shape=[
            jax.ShapeDtypeStruct((8, T), jnp.int32),
            jax.ShapeDtypeStruct((8, T), jnp.float32),
        ],
    )(x, gwp)


def _bin_a_kernel(ids_hbm, lane_hbm, rank_hbm, cnt_hbm,
                  eids_v, rank_v, cnt_v, shf_v, lane_v):
    wid = _wid()
    flatbase = wid * IPW
    pltpu.sync_copy(lane_hbm, lane_v)
    lane = lane_v[pl.ds(0, 16)]
    zero = jnp.zeros((16,), jnp.int32)
    _shf_init(shf_v, zero)
    pltpu.sync_copy(ids_hbm.at[flatbase // T, pl.ds(flatbase % T, IPW)], eids_v)
    nv = IPW // 16
    ranks = [zero for _ in range(nv)]
    counts = zero
    for e in range(E):
        run = zero
        for v in range(nv):
            ev = eids_v[pl.ds(v * 16, 16)]
            m = ev == e
            c = _scan16(jnp.where(m, 1, 0), shf_v)
            ranks[v] = jnp.where(m, run + c - 1, ranks[v])
            run = run + _splat_last(c, shf_v, lane)
        counts = jnp.where(lane == e, run, counts)
    for v in range(nv):
        rank_v[pl.ds(v * 16, 16)] = ranks[v]
    cnt_v[pl.ds(0, 16)] = counts
    pltpu.sync_copy(rank_v, rank_hbm.at[pl.ds(flatbase, IPW)])
    pltpu.sync_copy(cnt_v, cnt_hbm.at[pl.ds(wid * 16, 16)])


def _bin_a(ids, lane16):
    f = functools.partial(
        pl.kernel,
        mesh=_mesh(),
        out_type=[
            jax.ShapeDtypeStruct((N_ITEMS,), jnp.int32),
            jax.ShapeDtypeStruct((NW * 16,), jnp.int32),
        ],
        scratch_types=[
            pltpu.VMEM((IPW,), jnp.int32),
            pltpu.VMEM((IPW,), jnp.int32),
            pltpu.VMEM((16,), jnp.int32),
            pltpu.VMEM((48,), jnp.int32),
            pltpu.VMEM((16,), jnp.int32),
        ],
    )(_bin_a_kernel)
    return f(ids, lane16)


def _bin_b_kernel(ids_hbm, lane_hbm, rank_hbm, cnt_hbm, pos_hbm, bexp_hbm,
                  eids_v, rank_v, pos_v, allc_v, shf_v, lane_v, pfx_v, bexp_v):
    wid = _wid()
    flatbase = wid * IPW
    pltpu.sync_copy(lane_hbm, lane_v)
    lane = lane_v[pl.ds(0, 16)]
    zero = jnp.zeros((16,), jnp.int32)
    _shf_init(shf_v, zero)
    pltpu.sync_copy(ids_hbm.at[flatbase // T, pl.ds(flatbase % T, IPW)], eids_v)
    pltpu.sync_copy(rank_hbm.at[pl.ds(flatbase, IPW)], rank_v)
    pltpu.sync_copy(cnt_hbm, allc_v)
    pfx_v[pl.ds(0, 16)] = zero
    total = zero
    for w in range(NW):
        rowv = allc_v[pl.ds(w * 16, 16)]
        total = total + rowv

        @pl.when(w < wid)
        def _acc(rowv=rowv):
            pfx_v[pl.ds(0, 16)] = pfx_v[pl.ds(0, 16)] + rowv

    padded = jnp.bitwise_and(total + (BLK - 1), -BLK)
    off = _scan16(padded, shf_v) - padded
    base = off + pfx_v[pl.ds(0, 16)]
    base_splats = [_splat_at(base, shf_v, lane, e) for e in range(E)]
    nv = IPW // 16
    for v in range(nv):
        ev = eids_v[pl.ds(v * 16, 16)]
        b = zero
        for e in range(E):
            b = jnp.where(ev == e, base_splats[e], b)
        pos_v[pl.ds(v * 16, 16)] = b + rank_v[pl.ds(v * 16, 16)]
    pltpu.sync_copy(pos_v, pos_hbm.at[pl.ds(flatbase, IPW)])

    @pl.when(wid == 0)
    def _sched():
        off_splats = [_splat_at(off, shf_v, lane, e) for e in range(E)]
        for cq in range(4):
            bidx = lane + cq * 16
            bstart = bidx * BLK
            acc = jnp.full((16,), -1, jnp.int32)
            for e in range(E):
                acc = acc + jnp.where(bstart >= off_splats[e], 1, 0)
            acc = jnp.where(bidx >= G_PAD, E, acc)
            bexp_v[pl.ds(cq * 16, 16)] = acc
        pltpu.sync_copy(bexp_v, bexp_hbm)


def _bin_b(ids, lane16, rank, cnt):
    f = functools.partial(
        pl.kernel,
        mesh=_mesh(),
        out_type=[
            jax.ShapeDtypeStruct((N_ITEMS,), jnp.int32),
            jax.ShapeDtypeStruct((64,), jnp.int32),
        ],
        scratch_types=[
            pltpu.VMEM((IPW,), jnp.int32),
            pltpu.VMEM((IPW,), jnp.int32),
            pltpu.VMEM((IPW,), jnp.int32),
            pltpu.VMEM((NW * 16,), jnp.int32),
            pltpu.VMEM((48,), jnp.int32),
            pltpu.VMEM((16,), jnp.int32),
            pltpu.VMEM((16,), jnp.int32),
            pltpu.VMEM((64,), jnp.int32),
        ],
    )(_bin_b_kernel)
    return f(ids, lane16, rank, cnt)


DCHUNK = 32
NCHUNKS = IPW // DCHUNK


def _dispatch_kernel(x_hbm, pos_hbm, f_hbm, xs_hbm, wsw_hbm,
                     idx_v, rows_v, wv_v, wsw_v, shf_v, semi, sems):
    wid = _wid()
    flatbase = wid * IPW
    t0 = flatbase % T
    wrow = flatbase // T
    _shf_init(shf_v, jnp.zeros((16,), jnp.float32))
    for cc in range(NCHUNKS):
        b = cc % 2
        c1 = pltpu.make_async_copy(
            pos_hbm.at[pl.ds(flatbase + cc * DCHUNK, DCHUNK)], idx_v.at[b], semi)
        c2 = pltpu.make_async_copy(
            x_hbm.at[pl.ds(t0 + cc * DCHUNK, DCHUNK)], rows_v.at[b], semi)
        c3 = pltpu.make_async_copy(
            f_hbm.at[wrow, pl.ds(t0 + cc * DCHUNK, DCHUNK)], wv_v.at[b], semi)
        c1.start()
        c2.start()
        c3.start()
        c1.wait()
        c2.wait()
        c3.wait()
        for r in range(DCHUNK):
            vi = wv_v[b, pl.ds((r // 16) * 16, 16)]
            sh = _shift_down(vi, shf_v, r % 16) if r % 16 else vi
            wsw_v[b, r, pl.ds(0, 16)] = sh
        s1 = pltpu.make_async_copy(rows_v.at[b], xs_hbm.at[idx_v.at[b]], sems)
        s2 = pltpu.make_async_copy(wsw_v.at[b], wsw_hbm.at[idx_v.at[b]], sems)
        s1.start()
        s2.start()
        s1.wait()
        s2.wait()


def _dispatch(x, pos, fvals):
    f = functools.partial(
        pl.kernel,
        mesh=_mesh(),
        out_type=[
            jax.ShapeDtypeStruct((P_MAX, H), jnp.float32),
            jax.ShapeDtypeStruct((P_MAX, 128), jnp.float32),
        ],
        scratch_types=[
            pltpu.VMEM((2, DCHUNK), jnp.int32),
            pltpu.VMEM((2, DCHUNK, H), jnp.float32),
            pltpu.VMEM((2, DCHUNK), jnp.float32),
            pltpu.VMEM((2, DCHUNK, 128), jnp.float32),
            pltpu.VMEM((48,), jnp.float32),
            pltpu.SemaphoreType.DMA,
            pltpu.SemaphoreType.DMA,
        ],
    )(_dispatch_kernel)
    return f(x, pos, fvals)


def _shared_body(x_ref, wg_ref, wu_ref, wd_ref, sgw_ref, y_ref):
    xb = x_ref[...]
    hg = lax.dot_general(xb, wg_ref[...], (((1,), (1,)), ((), ())),
                         preferred_element_type=jnp.float32)
    hu = lax.dot_general(xb, wu_ref[...], (((1,), (1,)), ((), ())),
                         preferred_element_type=jnp.float32)
    ha = hg * _sigmoid(hg) * hu
    y = lax.dot_general(ha, wd_ref[...], (((1,), (1,)), ((), ())),
                        preferred_element_type=jnp.float32)
    gate = _sigmoid(lax.dot_general(xb, sgw_ref[...], (((1,), (1,)), ((), ())),
                                    preferred_element_type=jnp.float32))
    y_ref[...] = y * gate[:, 0:1]


def _mlp_shared(x, wsg, wsu, wsd, sgwp):
    return pl.pallas_call(
        _shared_body,
        grid=(T // BLK,),
        in_specs=[
            pl.BlockSpec((BLK, H), lambda g: (g, 0)),
            pl.BlockSpec((FF, H), lambda g: (0, 0)),
            pl.BlockSpec((FF, H), lambda g: (0, 0)),
            pl.BlockSpec((H, FF), lambda g: (0, 0)),
            pl.BlockSpec((128, H), lambda g: (0, 0)),
        ],
        out_specs=pl.BlockSpec((BLK, H), lambda g: (g, 0)),
        out_shape=jax.ShapeDtypeStruct((T, H), jnp.float32),
    )(x, wsg, wsu, wsd, sgwp)


def _routed_body(s_ref, xs_ref, wg_ref, wu_ref, wd_ref, wsw_ref, y_ref):
    xb = xs_ref[...]
    hg = lax.dot_general(xb, wg_ref[0], (((1,), (1,)), ((), ())),
                         preferred_element_type=jnp.float32)
    hu = lax.dot_general(xb, wu_ref[0], (((1,), (1,)), ((), ())),
                         preferred_element_type=jnp.float32)
    ha = hg * _sigmoid(hg) * hu
    y = lax.dot_general(ha, wd_ref[0], (((1,), (1,)), ((), ())),
                        preferred_element_type=jnp.float32)
    y_ref[...] = y * wsw_ref[:, 0:1]


def _mlp_routed(x_sorted, wg, wu, wd, bexp, wsw):
    grid_spec = pltpu.PrefetchScalarGridSpec(
        num_scalar_prefetch=1,
        grid=(G_PAD,),
        in_specs=[
            pl.BlockSpec((BLK, H), lambda g, s: (g, 0)),
            pl.BlockSpec((1, FF, H), lambda g, s: (s[g], 0, 0)),
            pl.BlockSpec((1, FF, H), lambda g, s: (s[g], 0, 0)),
            pl.BlockSpec((1, H, FF), lambda g, s: (s[g], 0, 0)),
            pl.BlockSpec((BLK, 128), lambda g, s: (g, 0)),
        ],
        out_specs=pl.BlockSpec((BLK, H), lambda g, s: (g, 0)),
    )
    return pl.pallas_call(
        _routed_body,
        grid_spec=grid_spec,
        out_shape=jax.ShapeDtypeStruct((P_MAX, H), jnp.float32),
    )(bexp, x_sorted, wg, wu, wd, wsw)


TOK_PER_CTILE = T // NW
CCHUNK = 8
NCC = TOK_PER_CTILE // CCHUNK


def _combine_kernel(pos_hbm, yr_hbm, ys_hbm, out_hbm,
                    p0_v, p1_v, y0_v, y1_v, ys_v, o_v, sem_a, sem_b):
    wid = _wid()
    tbase = wid * TOK_PER_CTILE

    def gathers(cc, half, sem):
        t0 = tbase + cc * CCHUNK
        boff = half * CCHUNK
        return [
            pltpu.make_async_copy(yr_hbm.at[p0_v.at[half]],
                                  y0_v.at[pl.ds(boff, CCHUNK)], sem),
            pltpu.make_async_copy(yr_hbm.at[p1_v.at[half]],
                                  y1_v.at[pl.ds(boff, CCHUNK)], sem),
            pltpu.make_async_copy(ys_hbm.at[pl.ds(t0, CCHUNK)],
                                  ys_v.at[pl.ds(boff, CCHUNK)], sem),
        ]

    def issue(cc, half, sem):
        t0 = tbase + cc * CCHUNK
        pltpu.sync_copy(pos_hbm.at[pl.ds(t0, CCHUNK)], p0_v.at[half])
        pltpu.sync_copy(pos_hbm.at[pl.ds(T + t0, CCHUNK)], p1_v.at[half])
        for g in gathers(cc, half, sem):
            g.start()

    issue(0, 0, sem_a)

    def body(cc, carry):
        pr = lax.rem(cc, 2)

        @pl.when(pr == 0)
        def _even():
            for g in gathers(cc, 0, sem_a):
                g.wait()

            @pl.when(cc < NCC - 1)
            def _():
                issue(cc + 1, 1, sem_b)

        @pl.when(pr == 1)
        def _odd():
            for g in gathers(cc, 1, sem_b):
                g.wait()

            @pl.when(cc < NCC - 1)
            def _():
                issue(cc + 1, 0, sem_a)

        boff = pr * CCHUNK
        for tt in range(CCHUNK):
            row = boff + tt
            for j in range(H // 16):
                sl = pl.ds(j * 16, 16)
                o_v[tt, sl] = ys_v[row, sl] + y0_v[row, sl] + y1_v[row, sl]
        t0 = tbase + cc * CCHUNK
        pltpu.sync_copy(o_v, out_hbm.at[pl.ds(t0, CCHUNK)])
        return carry

    lax.fori_loop(0, NCC, body, 0)


def _combine(pos, y_r, y_s):
    f = functools.partial(
        pl.kernel,
        mesh=_mesh(),
        out_type=jax.ShapeDtypeStruct((T, H), jnp.float32),
        scratch_types=[
            pltpu.VMEM((2, CCHUNK), jnp.int32),
            pltpu.VMEM((2, CCHUNK), jnp.int32),
            pltpu.VMEM((2 * CCHUNK, H), jnp.float32),
            pltpu.VMEM((2 * CCHUNK, H), jnp.float32),
            pltpu.VMEM((2 * CCHUNK, H), jnp.float32),
            pltpu.VMEM((CCHUNK, H), jnp.float32),
            pltpu.SemaphoreType.DMA,
            pltpu.SemaphoreType.DMA,
        ],
    )(_combine_kernel)
    return f(pos, y_r, y_s)


def kernel(hidden_states, Wg, Wu, Wd, Wsg, Wsu, Wsd, gate_w, shared_gate_w):
    B, S, _ = hidden_states.shape
    x = hidden_states.reshape(B * S, H)
    gwp = jnp.zeros((128, H), jnp.float32).at[:E].set(gate_w).at[E].set(shared_gate_w[0])
    sgwp = jnp.zeros((128, H), jnp.float32).at[0].set(shared_gate_w[0])
    lane16 = jnp.arange(16, dtype=jnp.int32)

    ids, fvals = _router(x, gwp)
    y_s = _mlp_shared(x, Wsg, Wsu, Wsd, sgwp)
    rank, cnt = _bin_a(ids, lane16)
    pos, bexp = _bin_b(ids, lane16, rank, cnt)
    x_sorted, wsw = _dispatch(x, pos, fvals)
    y_r = _mlp_routed(x_sorted, Wg, Wu, Wd, bexp, wsw)
    out = _combine(pos, y_r, y_s)
    return out.reshape(B, S, H)
